# baseline (speedup 1.0000x reference)
import jax
import jax.numpy as jnp
from jax.experimental import pallas as pl
from jax.experimental.pallas import tpu as pltpu

_CHUNK_S = 512
_NBUF = 4


def _pipeline_kernel(seq_hbm, pe_hbm, out_hbm,
                     seq_buf, pe_buf, seq_sem, pe_sem, out_sem):
    batch, seq_len, d_model = seq_hbm.shape
    nchunk = seq_len // _CHUNK_S

    def seq_copy(i, slot):
        return pltpu.make_async_copy(
            seq_hbm.at[:, pl.ds(i * _CHUNK_S, _CHUNK_S), :],
            seq_buf.at[slot], seq_sem.at[slot])

    def pe_copy(i, slot):
        return pltpu.make_async_copy(
            pe_hbm.at[pl.ds(i * _CHUNK_S, _CHUNK_S), :],
            pe_buf.at[slot], pe_sem.at[slot])

    half = _CHUNK_S // 2

    def out_copy(i, slot, h):
        return pltpu.make_async_copy(
            seq_buf.at[slot, :, pl.ds(h * half, half), :],
            out_hbm.at[:, pl.ds(i * _CHUNK_S + h * half, half), :],
            out_sem.at[slot])

    for i in range(min(_NBUF, nchunk)):
        seq_copy(i, i).start()
        pe_copy(i, i).start()

    for i in range(nchunk):
        slot = i % _NBUF
        seq_copy(i, slot).wait()
        pe_copy(i, slot).wait()
        seq_buf[slot, :, :half] = (seq_buf[slot, :, :half]
                                   + pe_buf[slot, :half][None, :, :])
        out_copy(i, slot, 0).start()
        seq_buf[slot, :, half:] = (seq_buf[slot, :, half:]
                                   + pe_buf[slot, half:][None, :, :])
        out_copy(i, slot, 1).start()
        nxt = i + _NBUF
        if nxt < nchunk:
            out_copy(nxt - _NBUF, slot, 0).wait()
            out_copy(nxt - _NBUF, slot, 1).wait()
            seq_copy(nxt, slot).start()
            pe_copy(nxt, slot).start()

    for i in range(max(nchunk - _NBUF, 0), nchunk):
        out_copy(i, i % _NBUF, 0).wait()
        out_copy(i, i % _NBUF, 1).wait()


def kernel(sequence, pe):
    batch, seq_len, d_model = sequence.shape
    pe2d = pe[0, :seq_len]

    out = pl.pallas_call(
        _pipeline_kernel,
        in_specs=[
            pl.BlockSpec(memory_space=pl.ANY),
            pl.BlockSpec(memory_space=pl.ANY),
        ],
        out_specs=pl.BlockSpec(memory_space=pl.ANY),
        out_shape=jax.ShapeDtypeStruct(sequence.shape, sequence.dtype),
        scratch_shapes=[
            pltpu.VMEM((_NBUF, batch, _CHUNK_S, d_model), jnp.float32),
            pltpu.VMEM((_NBUF, _CHUNK_S, d_model), jnp.float32),
            pltpu.SemaphoreType.DMA((_NBUF,)),
            pltpu.SemaphoreType.DMA((_NBUF,)),
            pltpu.SemaphoreType.DMA((_NBUF,)),
        ],
    )(sequence, pe2d)
    return out


# speedup vs baseline: 1.0312x; 1.0312x over previous
import jax
import jax.numpy as jnp
from jax.experimental import pallas as pl
from jax.experimental.pallas import tpu as pltpu

_CHUNK_S = 512
_NBUF = 4


def _pipeline_kernel(seq_hbm, pe_hbm, out_hbm,
                     seq_buf, pe_buf, seq_sem, pe_sem, out_sem):
    batch, seq_len, d_model = seq_hbm.shape
    nchunk = seq_len // _CHUNK_S

    def seq_copy(i, slot):
        return pltpu.make_async_copy(
            seq_hbm.at[:, pl.ds(i * _CHUNK_S, _CHUNK_S), :],
            seq_buf.at[slot], seq_sem.at[slot])

    def pe_copy(i, slot):
        return pltpu.make_async_copy(
            pe_hbm.at[pl.ds(i * _CHUNK_S, _CHUNK_S), :],
            pe_buf.at[slot], pe_sem.at[slot])

    def out_copy(i, slot):
        return pltpu.make_async_copy(
            seq_buf.at[slot],
            out_hbm.at[:, pl.ds(i * _CHUNK_S, _CHUNK_S), :],
            out_sem.at[slot])

    for i in range(min(_NBUF, nchunk)):
        seq_copy(i, i).start()
        pe_copy(i, i).start()

    for i in range(nchunk):
        slot = i % _NBUF
        seq_copy(i, slot).wait()
        pe_copy(i, slot).wait()
        seq_buf[slot] = seq_buf[slot] + pe_buf[slot][None, :, :]
        out_copy(i, slot).start()
        nxt = i + _NBUF
        if nxt < nchunk:
            out_copy(i, slot).wait()
            seq_copy(nxt, slot).start()
            pe_copy(nxt, slot).start()

    for i in range(max(nchunk - _NBUF, 0), nchunk):
        out_copy(i, i % _NBUF).wait()


def kernel(sequence, pe):
    batch, seq_len, d_model = sequence.shape
    pe2d = pe[0, :seq_len]

    out = pl.pallas_call(
        _pipeline_kernel,
        in_specs=[
            pl.BlockSpec(memory_space=pl.ANY),
            pl.BlockSpec(memory_space=pl.ANY),
        ],
        out_specs=pl.BlockSpec(memory_space=pl.ANY),
        out_shape=jax.ShapeDtypeStruct(sequence.shape, sequence.dtype),
        scratch_shapes=[
            pltpu.VMEM((_NBUF, batch, _CHUNK_S, d_model), jnp.float32),
            pltpu.VMEM((_NBUF, _CHUNK_S, d_model), jnp.float32),
            pltpu.SemaphoreType.DMA((_NBUF,)),
            pltpu.SemaphoreType.DMA((_NBUF,)),
            pltpu.SemaphoreType.DMA((_NBUF,)),
        ],
    )(sequence, pe2d)
    return out
